# baseline (speedup 1.0000x reference)
import jax
import jax.numpy as jnp
from jax import lax
from jax.experimental import pallas as pl
from jax.experimental.pallas import tpu as pltpu
from jax.experimental.pallas import tpu_sc as plsc

_B = 16384
_ED = 32
_NC = 2
_NS = 16
_NW = _NC * _NS
_BPW = _B // _NW

_NUM_GENDERS = 2
_NUM_OCC = 21
_NUM_GENRES = 18
_H = 128

_TB = 2048


def _sc_gather_body(user_hbm, movie_hbm, utab_hbm, mtab_hbm,
                    uemb_hbm, memb_hbm,
                    uidx_v, midx_v, urows_v, mrows_v, sem_u, sem_m):
    wid = lax.axis_index("s") * _NC + lax.axis_index("c")
    base = wid * _BPW
    pltpu.sync_copy(user_hbm.at[pl.ds(base, _BPW)], uidx_v)
    pltpu.sync_copy(movie_hbm.at[pl.ds(base, _BPW)], midx_v)
    cu = pltpu.async_copy(utab_hbm.at[uidx_v], urows_v, sem_u)
    cm = pltpu.async_copy(mtab_hbm.at[midx_v], mrows_v, sem_m)
    cu.wait()
    cm.wait()
    pltpu.sync_copy(urows_v, uemb_hbm.at[pl.ds(base, _BPW)])
    pltpu.sync_copy(mrows_v, memb_hbm.at[pl.ds(base, _BPW)])


import functools


@functools.cache
def _sc_gather():
    return pl.kernel(
        _sc_gather_body,
        out_type=(jax.ShapeDtypeStruct((_B, _ED), jnp.float32),
                  jax.ShapeDtypeStruct((_B, _ED), jnp.float32)),
        mesh=plsc.VectorSubcoreMesh(core_axis_name="c", subcore_axis_name="s",
                                    num_cores=_NC, num_subcores=_NS),
        scratch_types=[
            pltpu.VMEM((_BPW,), jnp.int32),
            pltpu.VMEM((_BPW,), jnp.int32),
            pltpu.VMEM((_BPW, _ED), jnp.float32),
            pltpu.VMEM((_BPW, _ED), jnp.float32),
            pltpu.SemaphoreType.DMA,
            pltpu.SemaphoreType.DMA,
        ],
        compiler_params=pltpu.CompilerParams(use_tc_tiling_on_sc=False),
    )


def _mlp_body(uemb, memb, gender, occ, genres, gtab, otab,
              wg, bg, w1, b1, w2, b2, out):
    w1r = w1[...]
    f32 = jnp.float32
    genre_w = jnp.dot(wg[...], w1r[128:160, :], preferred_element_type=f32)
    gt_w = jnp.dot(gtab[...], w1r[64:96, :], preferred_element_type=f32)
    ot_w = jnp.dot(otab[...], w1r[96:128, :], preferred_element_type=f32)
    bias = b1[...] + jnp.dot(bg[...], w1r[128:160, :], preferred_element_type=f32)

    g1h = (lax.broadcasted_iota(jnp.int32, (_TB, _NUM_GENDERS), 1)
           == gender[...]).astype(f32)
    o1h = (lax.broadcasted_iota(jnp.int32, (_TB, _NUM_OCC), 1)
           == occ[...]).astype(f32)

    h = (jnp.dot(uemb[...], w1r[0:32, :], preferred_element_type=f32)
         + jnp.dot(memb[...], w1r[32:64, :], preferred_element_type=f32)
         + jnp.dot(g1h, gt_w, preferred_element_type=f32)
         + jnp.dot(o1h, ot_w, preferred_element_type=f32)
         + jnp.dot(genres[...], genre_w, preferred_element_type=f32)
         + bias)
    h = jnp.maximum(h, 0.0)
    out[...] = jnp.dot(h, w2[...], preferred_element_type=f32) + b2[...]


def _mlp_call(uemb, memb, gender2d, occ2d, genres, gtab, otab,
              wg, bg2d, w1, b12d, w2, b22d):
    grid = (_B // _TB,)
    full = lambda i: (0, 0)
    return pl.pallas_call(
        _mlp_body,
        grid=grid,
        in_specs=[
            pl.BlockSpec((_TB, _ED), lambda i: (i, 0)),
            pl.BlockSpec((_TB, _ED), lambda i: (i, 0)),
            pl.BlockSpec((_TB, 1), lambda i: (i, 0)),
            pl.BlockSpec((_TB, 1), lambda i: (i, 0)),
            pl.BlockSpec((_TB, _NUM_GENRES), lambda i: (i, 0)),
            pl.BlockSpec((_NUM_GENDERS, _ED), full),
            pl.BlockSpec((_NUM_OCC, _ED), full),
            pl.BlockSpec((_NUM_GENRES, _ED), full),
            pl.BlockSpec((1, _ED), full),
            pl.BlockSpec((5 * _ED, _H), full),
            pl.BlockSpec((1, _H), full),
            pl.BlockSpec((_H, 1), full),
            pl.BlockSpec((1, 1), full),
        ],
        out_specs=pl.BlockSpec((_TB, 1), lambda i: (i, 0)),
        out_shape=jax.ShapeDtypeStruct((_B, 1), jnp.float32),
    )(uemb, memb, gender2d, occ2d, genres, gtab, otab,
      wg, bg2d, w1, b12d, w2, b22d)


def kernel(user, movie, gender, occupation, genres,
           user_table, movie_table, gender_table, occupation_table,
           W_genre, b_genre, W1, b1, W2, b2):
    user = user.astype(jnp.int32)
    movie = movie.astype(jnp.int32)
    uemb, memb = _sc_gather()(user, movie, user_table, movie_table)
    out = _mlp_call(
        uemb, memb,
        gender.astype(jnp.int32).reshape(_B, 1),
        occupation.astype(jnp.int32).reshape(_B, 1),
        genres.astype(jnp.float32),
        gender_table, occupation_table,
        W_genre, b_genre.reshape(1, _ED),
        W1, b1.reshape(1, _H), W2, b2.reshape(1, 1),
    )
    return out.reshape(_B)
